# baseline (speedup 1.0000x reference)
import functools

import jax
import jax.numpy as jnp
from jax import lax
from jax.experimental import pallas as pl
from jax.experimental.pallas import tpu as pltpu
from jax.experimental.pallas import tpu_sc as plsc

_NC = 2
_NS = 16
_NW = _NC * _NS
_K = 128
_D = 4
_DD = 6
_ROWS_B = 2000


def _sc_mesh():
    return plsc.VectorSubcoreMesh(core_axis_name="c", subcore_axis_name="s",
                                  num_cores=_NC, num_subcores=_NS)


_RZ = 632


def _init_rows(src_h, dst_h, sid, n):
    last = _NS - 1
    tail = n - last * _RZ

    @pl.when(sid < last)
    def _():
        off = pl.multiple_of(sid * _RZ, 8)
        pltpu.sync_copy(src_h.at[pl.ds(off, _RZ)], dst_h.at[pl.ds(off, _RZ)])

    @pl.when(sid == last)
    def _():
        off = pl.multiple_of(last * _RZ, 8)
        pltpu.sync_copy(src_h.at[pl.ds(off, tail)],
                        dst_h.at[pl.ds(off, tail)])


def _init_1d(src_h, dst_h, sid, owner):
    @pl.when(sid == owner)
    def _():
        pltpu.sync_copy(src_h, dst_h)


def _sc_degree(dst, zeros_n):
    n = zeros_n.shape[0]
    e = dst.shape[0]
    e_per = e // _NW
    nfull = e_per // _K
    tail = e_per - nfull * _K

    @functools.partial(
        pl.kernel,
        out_type=jax.ShapeDtypeStruct((_NC, n), jnp.float32),
        mesh=_sc_mesh(),
        scratch_types=[
            pltpu.VMEM((_DD, _K), jnp.int32),
            pltpu.VMEM((_K,), jnp.float32),
            pltpu.VMEM((tail,), jnp.int32) if tail else None,
            pltpu.VMEM_SHARED((n,), jnp.float32),
            pltpu.SemaphoreType.DMA((_DD,)),
            pltpu.SemaphoreType.DMA((_DD,)),
        ],
    )
    def deg_kernel(dst_h, zeros_h, out_h, di_r, ones_v, di_t, acc,
                   isem, ssem):
        cid = lax.axis_index("c")
        sid = lax.axis_index("s")
        wid = sid * _NC + cid
        base = wid * e_per
        for i in range(_K // 16):
            ones_v[pl.ds(i * 16, 16)] = jnp.full((16,), 1.0, jnp.float32)

        def fire_idx(j, b):
            off = pl.multiple_of(base + j * _K, 8)
            pltpu.async_copy(dst_h.at[pl.ds(off, _K)], di_r.at[b],
                             isem.at[b])

        def wait_idx(j, b):
            off = pl.multiple_of(base + j * _K, 8)
            pltpu.make_async_copy(dst_h.at[pl.ds(off, _K)], di_r.at[b],
                                  isem.at[b]).wait()

        def fire_sc(b):
            pltpu.async_copy(ones_v, acc.at[di_r.at[b]], ssem.at[b],
                             add=True)

        def wait_sc(b):
            pltpu.make_async_copy(ones_v, acc.at[di_r.at[b]],
                                  ssem.at[b]).wait()

        fire_idx(0, 0)
        fire_idx(1, 1)
        _init_1d(zeros_h, acc, sid, 0)
        plsc.subcore_barrier()

        def step(j, b):
            @pl.when(j >= 3)
            def _():
                wait_sc((b + 3) % _DD)

            wait_idx(j, b)
            fire_sc(b)

            @pl.when(j + 2 < nfull)
            def _():
                fire_idx(j + 2, (b + 2) % _DD)

        def body(i, carry):
            for u in range(_DD):
                step(i * _DD + u, u)
            return carry

        lax.fori_loop(0, nfull // _DD, body, 0)
        for j in range(nfull - nfull % _DD, nfull):
            step(jnp.int32(j), j % _DD)
        for j in range(max(0, nfull - 3), nfull):
            wait_sc(j % _DD)
        if tail:
            off = pl.multiple_of(base + nfull * _K, 8)
            pltpu.sync_copy(dst_h.at[pl.ds(off, tail)], di_t)
            pltpu.sync_copy(ones_v.at[pl.ds(0, tail)], acc.at[di_t],
                            add=True)
        plsc.subcore_barrier()
        _init_1d(acc, out_h.at[cid], sid, 0)

    return deg_kernel(dst, zeros_n)


def _sc_aggregate(table, src, dst, zeros_nd, dis=None, zeros_n=None):
    n, d = table.shape
    e = src.shape[0]
    e_per = e // _NW
    nfull = e_per // _K
    tail = e_per - nfull * _K
    with_s = dis is not None

    out_types = [jax.ShapeDtypeStruct((_NC, n, d), jnp.float32)]
    scratch = [
        pltpu.VMEM((_D, _K), jnp.int32),
        pltpu.VMEM((_D, _K), jnp.int32),
        pltpu.VMEM((2, _K, d), jnp.float32),
        pltpu.VMEM((tail,), jnp.int32) if tail else None,
        pltpu.VMEM((tail,), jnp.int32) if tail else None,
        pltpu.VMEM((tail, d), jnp.float32) if tail else None,
        pltpu.VMEM_SHARED((n, d), jnp.float32),
        pltpu.SemaphoreType.DMA((_D,)),
        pltpu.SemaphoreType.DMA((2,)),
        pltpu.SemaphoreType.DMA((2,)),
    ]
    if with_s:
        out_types.append(jax.ShapeDtypeStruct((_NC, n), jnp.float32))
        scratch += [
            pltpu.VMEM((2, _K), jnp.float32),
            pltpu.VMEM((tail,), jnp.float32) if tail else None,
            pltpu.VMEM_SHARED((n,), jnp.float32),
            pltpu.SemaphoreType.DMA((2,)),
            pltpu.SemaphoreType.DMA((2,)),
        ]

    @functools.partial(
        pl.kernel,
        out_type=tuple(out_types),
        mesh=_sc_mesh(),
        scratch_types=scratch,
    )
    def agg_kernel(*refs):
        if with_s:
            (table_h, src_h, dst_h, zeros2_h, dis_h, zeros1_h,
             out_h, s_out_h,
             si_r, di_r, rows_r, si_t, di_t, rows_t, acc,
             isem, gsem, ssem,
             val_r, val_t, s_acc, vgsem, sssem) = refs
        else:
            (table_h, src_h, dst_h, zeros2_h,
             out_h,
             si_r, di_r, rows_r, si_t, di_t, rows_t, acc,
             isem, gsem, ssem) = refs
        cid = lax.axis_index("c")
        sid = lax.axis_index("s")
        wid = sid * _NC + cid
        base = wid * e_per

        def fire_idx(j, b):
            off = pl.multiple_of(base + j * _K, 8)
            pltpu.async_copy(src_h.at[pl.ds(off, _K)], si_r.at[b],
                             isem.at[b])
            pltpu.async_copy(dst_h.at[pl.ds(off, _K)], di_r.at[b],
                             isem.at[b])

        def wait_idx(j, b):
            off = pl.multiple_of(base + j * _K, 8)
            pltpu.make_async_copy(src_h.at[pl.ds(off, _K)], si_r.at[b],
                                  isem.at[b]).wait()
            pltpu.make_async_copy(dst_h.at[pl.ds(off, _K)], di_r.at[b],
                                  isem.at[b]).wait()

        def fire_gather(ib, rb):
            pltpu.async_copy(table_h.at[si_r.at[ib]], rows_r.at[rb],
                             gsem.at[rb])
            if with_s:
                pltpu.async_copy(dis_h.at[di_r.at[ib]], val_r.at[rb],
                                 vgsem.at[rb])

        def wait_gather(ib, rb):
            pltpu.make_async_copy(table_h.at[si_r.at[ib]], rows_r.at[rb],
                                  gsem.at[rb]).wait()
            if with_s:
                pltpu.make_async_copy(dis_h.at[di_r.at[ib]], val_r.at[rb],
                                      vgsem.at[rb]).wait()

        def fire_sc(ib, rb):
            pltpu.async_copy(rows_r.at[rb], acc.at[di_r.at[ib]],
                             ssem.at[rb], add=True)
            if with_s:
                pltpu.async_copy(val_r.at[rb], s_acc.at[si_r.at[ib]],
                                 sssem.at[rb], add=True)

        def wait_sc(ib, rb):
            pltpu.make_async_copy(rows_r.at[rb], acc.at[di_r.at[ib]],
                                  ssem.at[rb]).wait()
            if with_s:
                pltpu.make_async_copy(val_r.at[rb], s_acc.at[si_r.at[ib]],
                                      sssem.at[rb]).wait()

        fire_idx(0, 0)
        fire_idx(1, 1)
        _init_rows(zeros2_h, acc, sid, n)
        if with_s:
            _init_1d(zeros1_h, s_acc, sid, _NS - 1)
        wait_idx(0, 0)
        fire_gather(0, 0)
        plsc.subcore_barrier()

        def step(j, ib, rb):
            @pl.when(j >= 1)
            def _():
                wait_sc((ib + _D - 1) % _D, 1 - rb)

            @pl.when(j + 1 < nfull)
            def _():
                wait_idx(j + 1, (ib + 1) % _D)
                fire_gather((ib + 1) % _D, 1 - rb)

            wait_gather(ib, rb)
            fire_sc(ib, rb)

            @pl.when(j + 2 < nfull)
            def _():
                fire_idx(j + 2, (ib + 2) % _D)

        def body(i, carry):
            for u in range(_D):
                step(i * _D + u, u, u % 2)
            return carry

        lax.fori_loop(0, nfull // _D, body, 0)
        for j in range(nfull - nfull % _D, nfull):
            step(jnp.int32(j), j % _D, j % 2)
        wait_sc((nfull - 1) % _D, (nfull - 1) % 2)
        if tail:
            off = pl.multiple_of(base + nfull * _K, 8)
            pltpu.sync_copy(src_h.at[pl.ds(off, tail)], si_t)
            pltpu.sync_copy(dst_h.at[pl.ds(off, tail)], di_t)
            pltpu.async_copy(table_h.at[si_t], rows_t, gsem.at[0]).wait()
            pltpu.sync_copy(rows_t, acc.at[di_t], add=True)
            if with_s:
                pltpu.async_copy(dis_h.at[di_t], val_t, vgsem.at[0]).wait()
                pltpu.sync_copy(val_t, s_acc.at[si_t], add=True)
        plsc.subcore_barrier()

        _init_rows(acc, out_h.at[cid], sid, n)
        if with_s:
            _init_1d(s_acc, s_out_h.at[cid], sid, _NS - 1)

    if with_s:
        return agg_kernel(table, src, dst, zeros_nd, dis, zeros_n)
    return agg_kernel(table, src, dst, zeros_nd)[0]


def _tc_matmul(x, w1):
    n, d_in = x.shape
    d_h = w1.shape[1]
    nb = n // _ROWS_B

    def body(x_ref, w_ref, h_ref):
        h_ref[...] = jnp.dot(x_ref[...], w_ref[...],
                             preferred_element_type=jnp.float32)

    return pl.pallas_call(
        body,
        grid=(nb,),
        in_specs=[
            pl.BlockSpec((_ROWS_B, d_in), lambda i: (i, 0)),
            pl.BlockSpec((d_in, d_h), lambda i: (0, 0)),
        ],
        out_specs=pl.BlockSpec((_ROWS_B, d_h), lambda i: (i, 0)),
        out_shape=jax.ShapeDtypeStruct((n, d_h), jnp.float32),
    )(x, w1)


def _tc_prep(cnt_t, h):
    n, d_h = h.shape
    nb = n // _ROWS_B

    def body(cnt_ref, h_ref, dis_ref, table_ref):
        c = cnt_ref[...]
        deg = c[:, 0:1] + c[:, 1:2] + 1.0
        dis = lax.rsqrt(deg)
        dis_ref[...] = dis
        table_ref[...] = dis * h_ref[...]

    return pl.pallas_call(
        body,
        grid=(nb,),
        in_specs=[
            pl.BlockSpec((_ROWS_B, 2), lambda i: (i, 0)),
            pl.BlockSpec((_ROWS_B, d_h), lambda i: (i, 0)),
        ],
        out_specs=[
            pl.BlockSpec((_ROWS_B, 1), lambda i: (i, 0)),
            pl.BlockSpec((_ROWS_B, d_h), lambda i: (i, 0)),
        ],
        out_shape=[
            jax.ShapeDtypeStruct((n, 1), jnp.float32),
            jax.ShapeDtypeStruct((n, d_h), jnp.float32),
        ],
    )(cnt_t, h)


def _tc_layer(partials, table, dis2, b_row, w_next):
    n, d = table.shape
    d_next = w_next.shape[1]
    nb = n // _ROWS_B

    def body(p_ref, t_ref, dis_ref, b_ref, w_ref, out_ref):
        p = p_ref[0] + p_ref[1]
        dis = dis_ref[...]
        h = jnp.maximum(dis * (p + t_ref[...]) + b_ref[...], 0.0)
        out_ref[...] = dis * jnp.dot(h, w_ref[...],
                                     preferred_element_type=jnp.float32)

    return pl.pallas_call(
        body,
        grid=(nb,),
        in_specs=[
            pl.BlockSpec((2, _ROWS_B, d), lambda i: (0, i, 0)),
            pl.BlockSpec((_ROWS_B, d), lambda i: (i, 0)),
            pl.BlockSpec((_ROWS_B, 1), lambda i: (i, 0)),
            pl.BlockSpec((1, d), lambda i: (0, 0)),
            pl.BlockSpec((d, d_next), lambda i: (0, 0)),
        ],
        out_specs=pl.BlockSpec((_ROWS_B, d_next), lambda i: (i, 0)),
        out_shape=jax.ShapeDtypeStruct((n, d_next), jnp.float32),
    )(partials, table, dis2, b_row, w_next)


def _tc_final(partials, table, dis2, s_t, b2_row, w3, b3_row, wc, bc_row):
    n, d = table.shape
    d_out = wc.shape[1]
    nb = n // _ROWS_B

    def body(p_ref, t_ref, dis_ref, s_ref, b2_ref, w3_ref, b3_ref, wc_ref,
             bc_ref, t_acc_ref, logits_ref):
        i = pl.program_id(0)
        dis = dis_ref[...]
        p = p_ref[0] + p_ref[1]
        h2 = jnp.maximum(dis * (p + t_ref[...]) + b2_ref[...], 0.0)
        s = s_ref[:, 0:1] + s_ref[:, 1:2]
        c = dis * (dis + s)
        contrib = jnp.sum(c * h2, axis=0, keepdims=True)

        @pl.when(i == 0)
        def _():
            t_acc_ref[...] = jnp.zeros_like(t_acc_ref)

        t_acc_ref[...] += contrib

        @pl.when(i == nb - 1)
        def _():
            t = t_acc_ref[...] * (1.0 / n)
            g = jnp.dot(t, w3_ref[...],
                        preferred_element_type=jnp.float32) + b3_ref[...]
            logits_ref[...] = jnp.dot(g, wc_ref[...],
                                      preferred_element_type=jnp.float32) \
                + bc_ref[...]

    _, logits = pl.pallas_call(
        body,
        grid=(nb,),
        in_specs=[
            pl.BlockSpec((2, _ROWS_B, d), lambda i: (0, i, 0)),
            pl.BlockSpec((_ROWS_B, d), lambda i: (i, 0)),
            pl.BlockSpec((_ROWS_B, 1), lambda i: (i, 0)),
            pl.BlockSpec((_ROWS_B, 2), lambda i: (i, 0)),
            pl.BlockSpec((1, d), lambda i: (0, 0)),
            pl.BlockSpec((d, d), lambda i: (0, 0)),
            pl.BlockSpec((1, d), lambda i: (0, 0)),
            pl.BlockSpec((d, d_out), lambda i: (0, 0)),
            pl.BlockSpec((1, d_out), lambda i: (0, 0)),
        ],
        out_specs=[
            pl.BlockSpec((1, d), lambda i: (0, 0)),
            pl.BlockSpec((1, d_out), lambda i: (0, 0)),
        ],
        out_shape=[
            jax.ShapeDtypeStruct((1, d), jnp.float32),
            jax.ShapeDtypeStruct((1, d_out), jnp.float32),
        ],
    )(partials, table, dis2, s_t, b2_row, w3, b3_row, wc, bc_row)
    return logits


def kernel(x, edge_index, W1, b1, W2, b2, W3, b3, Wc, bc):
    n = x.shape[0]
    d_h = W1.shape[1]
    src = edge_index[0]
    dst = edge_index[1]
    zeros_n = jnp.zeros((n,), jnp.float32)
    zeros_nd = jnp.zeros((n, d_h), jnp.float32)

    h_raw = _tc_matmul(x, W1)
    cnt_p = _sc_degree(dst, zeros_n)
    dis2, table1 = _tc_prep(cnt_p.T, h_raw)
    p1 = _sc_aggregate(table1, src, dst, zeros_nd)
    table2 = _tc_layer(p1, table1, dis2, b1.reshape(1, -1), W2)
    p2, s_p = _sc_aggregate(table2, src, dst, zeros_nd,
                            dis=dis2.reshape(-1), zeros_n=zeros_n)
    logits = _tc_final(p2, table2, dis2, s_p.T, b2.reshape(1, -1),
                       W3, b3.reshape(1, -1), Wc, bc.reshape(1, -1))
    return logits


# speedup vs baseline: 1.0067x; 1.0067x over previous
import functools

import jax
import jax.numpy as jnp
from jax import lax
from jax.experimental import pallas as pl
from jax.experimental.pallas import tpu as pltpu
from jax.experimental.pallas import tpu_sc as plsc

_NC = 2
_NS = 16
_NW = _NC * _NS
_K = 128
_D = 4
_DD = 6
_ROWS_B = 2000
_RZ = 632


def _sc_mesh():
    return plsc.VectorSubcoreMesh(core_axis_name="c", subcore_axis_name="s",
                                  num_cores=_NC, num_subcores=_NS)


def _init_rows(src_h, dst_h, sid, n):
    last = _NS - 1
    tail = n - last * _RZ

    @pl.when(sid < last)
    def _():
        off = pl.multiple_of(sid * _RZ, 8)
        pltpu.sync_copy(src_h.at[pl.ds(off, _RZ)], dst_h.at[pl.ds(off, _RZ)])

    @pl.when(sid == last)
    def _():
        off = pl.multiple_of(last * _RZ, 8)
        pltpu.sync_copy(src_h.at[pl.ds(off, tail)],
                        dst_h.at[pl.ds(off, tail)])


def _init_1d(src_h, dst_h, sid, owner):
    @pl.when(sid == owner)
    def _():
        pltpu.sync_copy(src_h, dst_h)


def _sc_degree(ei1, zeros_n):
    n = zeros_n.shape[0]
    e = ei1.shape[0] // 2
    e_per = e // _NW
    nfull = e_per // _K
    tail = e_per - nfull * _K

    @functools.partial(
        pl.kernel,
        out_type=jax.ShapeDtypeStruct((_NC, n), jnp.float32),
        mesh=_sc_mesh(),
        scratch_types=[
            pltpu.VMEM((_DD, _K), jnp.int32),
            pltpu.VMEM((_K,), jnp.float32),
            pltpu.VMEM((tail,), jnp.int32) if tail else None,
            pltpu.VMEM_SHARED((n,), jnp.float32),
            pltpu.SemaphoreType.DMA((_DD,)),
            pltpu.SemaphoreType.DMA((_DD,)),
        ],
    )
    def deg_kernel(ei_h, zeros_h, out_h, di_r, ones_v, di_t, acc,
                   isem, ssem):
        cid = lax.axis_index("c")
        sid = lax.axis_index("s")
        wid = sid * _NC + cid
        base = wid * e_per
        for i in range(_K // 16):
            ones_v[pl.ds(i * 16, 16)] = jnp.full((16,), 1.0, jnp.float32)

        def fire_idx(j, b):
            off = pl.multiple_of(e + base + j * _K, 8)
            pltpu.async_copy(ei_h.at[pl.ds(off, _K)], di_r.at[b],
                             isem.at[b])

        def wait_idx(j, b):
            off = pl.multiple_of(e + base + j * _K, 8)
            pltpu.make_async_copy(ei_h.at[pl.ds(off, _K)], di_r.at[b],
                                  isem.at[b]).wait()

        def fire_sc(b):
            pltpu.async_copy(ones_v, acc.at[di_r.at[b]], ssem.at[b],
                             add=True)

        def wait_sc(b):
            pltpu.make_async_copy(ones_v, acc.at[di_r.at[b]],
                                  ssem.at[b]).wait()

        fire_idx(0, 0)
        fire_idx(1, 1)
        _init_1d(zeros_h, acc, sid, 0)
        plsc.subcore_barrier()

        def step(j, b):
            @pl.when(j >= 3)
            def _():
                wait_sc((b + 3) % _DD)

            wait_idx(j, b)
            fire_sc(b)

            @pl.when(j + 2 < nfull)
            def _():
                fire_idx(j + 2, (b + 2) % _DD)

        def body(i, carry):
            for u in range(_DD):
                step(i * _DD + u, u)
            return carry

        lax.fori_loop(0, nfull // _DD, body, 0)
        for j in range(nfull - nfull % _DD, nfull):
            step(jnp.int32(j), j % _DD)
        for j in range(max(0, nfull - 3), nfull):
            wait_sc(j % _DD)
        if tail:
            off = pl.multiple_of(e + base + nfull * _K, 8)
            pltpu.sync_copy(ei_h.at[pl.ds(off, tail)], di_t)
            pltpu.sync_copy(ones_v.at[pl.ds(0, tail)], acc.at[di_t],
                            add=True)
        plsc.subcore_barrier()
        _init_1d(acc, out_h.at[cid], sid, 0)

    return deg_kernel(ei1, zeros_n)


def _sc_aggregate(table, ei1, zeros_nd, dis=None, zeros_n=None):
    n, d = table.shape
    e = ei1.shape[0] // 2
    e_per = e // _NW
    nfull = e_per // _K
    tail = e_per - nfull * _K
    with_s = dis is not None

    out_types = [jax.ShapeDtypeStruct((_NC, n, d), jnp.float32)]
    scratch = [
        pltpu.VMEM((_D, _K), jnp.int32),
        pltpu.VMEM((_D, _K), jnp.int32),
        pltpu.VMEM((2, _K, d), jnp.float32),
        pltpu.VMEM((tail,), jnp.int32) if tail else None,
        pltpu.VMEM((tail,), jnp.int32) if tail else None,
        pltpu.VMEM((tail, d), jnp.float32) if tail else None,
        pltpu.VMEM_SHARED((n, d), jnp.float32),
        pltpu.SemaphoreType.DMA((_D,)),
        pltpu.SemaphoreType.DMA((2,)),
        pltpu.SemaphoreType.DMA((2,)),
    ]
    if with_s:
        out_types.append(jax.ShapeDtypeStruct((_NC, n), jnp.float32))
        scratch += [
            pltpu.VMEM((2, _K), jnp.float32),
            pltpu.VMEM((tail,), jnp.float32) if tail else None,
            pltpu.VMEM_SHARED((n,), jnp.float32),
            pltpu.SemaphoreType.DMA((2,)),
            pltpu.SemaphoreType.DMA((2,)),
        ]

    @functools.partial(
        pl.kernel,
        out_type=tuple(out_types),
        mesh=_sc_mesh(),
        scratch_types=scratch,
    )
    def agg_kernel(*refs):
        if with_s:
            (table_h, ei_h, zeros2_h, dis_h, zeros1_h,
             out_h, s_out_h,
             si_r, di_r, rows_r, si_t, di_t, rows_t, acc,
             isem, gsem, ssem,
             val_r, val_t, s_acc, vgsem, sssem) = refs
        else:
            (table_h, ei_h, zeros2_h,
             out_h,
             si_r, di_r, rows_r, si_t, di_t, rows_t, acc,
             isem, gsem, ssem) = refs
        cid = lax.axis_index("c")
        sid = lax.axis_index("s")
        wid = sid * _NC + cid
        base = wid * e_per

        def fire_idx(j, b):
            off = pl.multiple_of(base + j * _K, 8)
            pltpu.async_copy(ei_h.at[pl.ds(off, _K)], si_r.at[b],
                             isem.at[b])
            pltpu.async_copy(ei_h.at[pl.ds(e + off, _K)], di_r.at[b],
                             isem.at[b])

        def wait_idx(j, b):
            off = pl.multiple_of(base + j * _K, 8)
            pltpu.make_async_copy(ei_h.at[pl.ds(off, _K)], si_r.at[b],
                                  isem.at[b]).wait()
            pltpu.make_async_copy(ei_h.at[pl.ds(e + off, _K)], di_r.at[b],
                                  isem.at[b]).wait()

        def fire_gather(ib, rb):
            pltpu.async_copy(table_h.at[si_r.at[ib]], rows_r.at[rb],
                             gsem.at[rb])
            if with_s:
                pltpu.async_copy(dis_h.at[di_r.at[ib]], val_r.at[rb],
                                 vgsem.at[rb])

        def wait_gather(ib, rb):
            pltpu.make_async_copy(table_h.at[si_r.at[ib]], rows_r.at[rb],
                                  gsem.at[rb]).wait()
            if with_s:
                pltpu.make_async_copy(dis_h.at[di_r.at[ib]],
                                      val_r.at[rb], vgsem.at[rb]).wait()

        def fire_sc(ib, rb):
            pltpu.async_copy(rows_r.at[rb], acc.at[di_r.at[ib]],
                             ssem.at[rb], add=True)
            if with_s:
                pltpu.async_copy(val_r.at[rb], s_acc.at[si_r.at[ib]],
                                 sssem.at[rb], add=True)

        def wait_sc(ib, rb):
            pltpu.make_async_copy(rows_r.at[rb], acc.at[di_r.at[ib]],
                                  ssem.at[rb]).wait()
            if with_s:
                pltpu.make_async_copy(val_r.at[rb], s_acc.at[si_r.at[ib]],
                                      sssem.at[rb]).wait()

        fire_idx(0, 0)
        fire_idx(1, 1)

        @pl.when(cid == 0)
        def _():
            _init_rows(table_h, acc, sid, n)

        @pl.when(cid == 1)
        def _():
            _init_rows(zeros2_h, acc, sid, n)

        if with_s:
            _init_1d(zeros1_h, s_acc, sid, _NS - 1)
        wait_idx(0, 0)
        fire_gather(0, 0)
        plsc.subcore_barrier()

        def step(j, ib, rb):
            @pl.when(j >= 1)
            def _():
                wait_sc((ib + _D - 1) % _D, 1 - rb)

            @pl.when(j + 1 < nfull)
            def _():
                wait_idx(j + 1, (ib + 1) % _D)
                fire_gather((ib + 1) % _D, 1 - rb)

            wait_gather(ib, rb)
            fire_sc(ib, rb)

            @pl.when(j + 2 < nfull)
            def _():
                fire_idx(j + 2, (ib + 2) % _D)

        def body(i, carry):
            for u in range(_D):
                step(i * _D + u, u, u % 2)
            return carry

        lax.fori_loop(0, nfull // _D, body, 0)
        for j in range(nfull - nfull % _D, nfull):
            step(jnp.int32(j), j % _D, j % 2)
        wait_sc((nfull - 1) % _D, (nfull - 1) % 2)
        if tail:
            off = pl.multiple_of(base + nfull * _K, 8)
            pltpu.sync_copy(ei_h.at[pl.ds(off, tail)], si_t)
            pltpu.sync_copy(ei_h.at[pl.ds(e + off, tail)], di_t)
            pltpu.async_copy(table_h.at[si_t], rows_t, gsem.at[0]).wait()
            pltpu.sync_copy(rows_t, acc.at[di_t], add=True)
            if with_s:
                pltpu.async_copy(dis_h.at[di_t], val_t,
                                 vgsem.at[0]).wait()
                pltpu.sync_copy(val_t, s_acc.at[si_t], add=True)
        plsc.subcore_barrier()

        _init_rows(acc, out_h.at[cid], sid, n)
        if with_s:
            _init_1d(s_acc, s_out_h.at[cid], sid, _NS - 1)

    if with_s:
        return agg_kernel(table, ei1, zeros_nd, dis, zeros_n)
    return agg_kernel(table, ei1, zeros_nd)[0]


def _tc_matmul(x, w1):
    n, d_in = x.shape
    d_h = w1.shape[1]
    nb = n // _ROWS_B

    def body(x_ref, w_ref, h_ref):
        h_ref[...] = jnp.dot(x_ref[...], w_ref[...],
                             preferred_element_type=jnp.float32)

    return pl.pallas_call(
        body,
        grid=(nb,),
        in_specs=[
            pl.BlockSpec((_ROWS_B, d_in), lambda i: (i, 0)),
            pl.BlockSpec((d_in, d_h), lambda i: (0, 0)),
        ],
        out_specs=pl.BlockSpec((_ROWS_B, d_h), lambda i: (i, 0)),
        out_shape=jax.ShapeDtypeStruct((n, d_h), jnp.float32),
    )(x, w1)


def _tc_prep(cnt_t, h):
    n, d_h = h.shape
    nb = n // _ROWS_B

    def body(cnt_ref, h_ref, dis_ref, table_ref):
        c = cnt_ref[...]
        deg = c[:, 0:1] + c[:, 1:2] + 1.0
        dis = lax.rsqrt(deg)
        dis_ref[...] = dis
        table_ref[...] = dis * h_ref[...]

    return pl.pallas_call(
        body,
        grid=(nb,),
        in_specs=[
            pl.BlockSpec((_ROWS_B, 2), lambda i: (i, 0)),
            pl.BlockSpec((_ROWS_B, d_h), lambda i: (i, 0)),
        ],
        out_specs=[
            pl.BlockSpec((_ROWS_B, 1), lambda i: (i, 0)),
            pl.BlockSpec((_ROWS_B, d_h), lambda i: (i, 0)),
        ],
        out_shape=[
            jax.ShapeDtypeStruct((n, 1), jnp.float32),
            jax.ShapeDtypeStruct((n, d_h), jnp.float32),
        ],
    )(cnt_t, h)


def _tc_layer(partials, dis2, b_row, w_next):
    n = partials.shape[1]
    d = partials.shape[2]
    d_next = w_next.shape[1]
    nb = n // _ROWS_B

    def body(p_ref, dis_ref, b_ref, w_ref, out_ref):
        p = p_ref[0] + p_ref[1]
        dis = dis_ref[...]
        h = jnp.maximum(dis * p + b_ref[...], 0.0)
        out_ref[...] = dis * jnp.dot(h, w_ref[...],
                                     preferred_element_type=jnp.float32)

    return pl.pallas_call(
        body,
        grid=(nb,),
        in_specs=[
            pl.BlockSpec((2, _ROWS_B, d), lambda i: (0, i, 0)),
            pl.BlockSpec((_ROWS_B, 1), lambda i: (i, 0)),
            pl.BlockSpec((1, d), lambda i: (0, 0)),
            pl.BlockSpec((d, d_next), lambda i: (0, 0)),
        ],
        out_specs=pl.BlockSpec((_ROWS_B, d_next), lambda i: (i, 0)),
        out_shape=jax.ShapeDtypeStruct((n, d_next), jnp.float32),
    )(partials, dis2, b_row, w_next)


def _tc_final(partials, dis2, s_t, b2_row, w3, b3_row, wc, bc_row):
    n = partials.shape[1]
    d = partials.shape[2]
    d_out = wc.shape[1]
    nb = n // _ROWS_B

    def body(p_ref, dis_ref, s_ref, b2_ref, w3_ref, b3_ref, wc_ref,
             bc_ref, t_acc_ref, logits_ref):
        i = pl.program_id(0)
        dis = dis_ref[...]
        p = p_ref[0] + p_ref[1]
        h2 = jnp.maximum(dis * p + b2_ref[...], 0.0)
        s = s_ref[:, 0:1] + s_ref[:, 1:2]
        c = dis * (dis + s)
        contrib = jnp.sum(c * h2, axis=0, keepdims=True)

        @pl.when(i == 0)
        def _():
            t_acc_ref[...] = jnp.zeros_like(t_acc_ref)

        t_acc_ref[...] += contrib

        @pl.when(i == nb - 1)
        def _():
            t = t_acc_ref[...] * (1.0 / n)
            g = jnp.dot(t, w3_ref[...],
                        preferred_element_type=jnp.float32) + b3_ref[...]
            logits_ref[...] = jnp.dot(g, wc_ref[...],
                                      preferred_element_type=jnp.float32) \
                + bc_ref[...]

    _, logits = pl.pallas_call(
        body,
        grid=(nb,),
        in_specs=[
            pl.BlockSpec((2, _ROWS_B, d), lambda i: (0, i, 0)),
            pl.BlockSpec((_ROWS_B, 1), lambda i: (i, 0)),
            pl.BlockSpec((_ROWS_B, 2), lambda i: (i, 0)),
            pl.BlockSpec((1, d), lambda i: (0, 0)),
            pl.BlockSpec((d, d), lambda i: (0, 0)),
            pl.BlockSpec((1, d), lambda i: (0, 0)),
            pl.BlockSpec((d, d_out), lambda i: (0, 0)),
            pl.BlockSpec((1, d_out), lambda i: (0, 0)),
        ],
        out_specs=[
            pl.BlockSpec((1, d), lambda i: (0, 0)),
            pl.BlockSpec((1, d_out), lambda i: (0, 0)),
        ],
        out_shape=[
            jax.ShapeDtypeStruct((1, d), jnp.float32),
            jax.ShapeDtypeStruct((1, d_out), jnp.float32),
        ],
    )(partials, dis2, s_t, b2_row, w3, b3_row, wc, bc_row)
    return logits


def kernel(x, edge_index, W1, b1, W2, b2, W3, b3, Wc, bc):
    n = x.shape[0]
    d_h = W1.shape[1]
    zeros_n = jnp.zeros((n,), jnp.float32)
    zeros_nd = jnp.zeros((n, d_h), jnp.float32)

    ei1 = edge_index.reshape(-1)
    h_raw = _tc_matmul(x, W1)
    cnt_p = _sc_degree(ei1, zeros_n)
    dis2, table1 = _tc_prep(cnt_p.T, h_raw)
    p1 = _sc_aggregate(table1, ei1, zeros_nd)
    table2 = _tc_layer(p1, dis2, b1.reshape(1, -1), W2)
    p2, s_p = _sc_aggregate(table2, ei1, zeros_nd,
                            dis=dis2.reshape(-1), zeros_n=zeros_n)
    logits = _tc_final(p2, dis2, s_p.T, b2.reshape(1, -1),
                       W3, b3.reshape(1, -1), Wc, bc.reshape(1, -1))
    return logits


# speedup vs baseline: 1.0602x; 1.0531x over previous
import functools

import jax
import jax.numpy as jnp
from jax import lax
from jax.experimental import pallas as pl
from jax.experimental.pallas import tpu as pltpu
from jax.experimental.pallas import tpu_sc as plsc

_NC = 2
_NS = 16
_NW = _NC * _NS
_K = 128
_D = 4
_DD = 6
_ROWS_B = 2000
_RZ = 632


def _sc_mesh():
    return plsc.VectorSubcoreMesh(core_axis_name="c", subcore_axis_name="s",
                                  num_cores=_NC, num_subcores=_NS)


def _init_rows(src_h, dst_h, sid, n):
    last = _NS - 1
    tail = n - last * _RZ

    @pl.when(sid < last)
    def _():
        off = pl.multiple_of(sid * _RZ, 8)
        pltpu.sync_copy(src_h.at[pl.ds(off, _RZ)], dst_h.at[pl.ds(off, _RZ)])

    @pl.when(sid == last)
    def _():
        off = pl.multiple_of(last * _RZ, 8)
        pltpu.sync_copy(src_h.at[pl.ds(off, tail)],
                        dst_h.at[pl.ds(off, tail)])


def _init_1d(src_h, dst_h, sid, owner):
    @pl.when(sid == owner)
    def _():
        pltpu.sync_copy(src_h, dst_h)


def _sc_degree(ei1, zeros_n):
    n = zeros_n.shape[0]
    e = ei1.shape[0] // 2
    e_per = e // _NW
    nfull = e_per // _K
    tail = e_per - nfull * _K

    @functools.partial(
        pl.kernel,
        out_type=jax.ShapeDtypeStruct((_NC, n), jnp.float32),
        mesh=_sc_mesh(),
        scratch_types=[
            pltpu.VMEM((_DD, _K), jnp.int32),
            pltpu.VMEM((_K,), jnp.float32),
            pltpu.VMEM((tail,), jnp.int32) if tail else None,
            pltpu.VMEM_SHARED((n,), jnp.float32),
            pltpu.SemaphoreType.DMA((_DD,)),
            pltpu.SemaphoreType.DMA((_DD,)),
        ],
    )
    def deg_kernel(ei_h, zeros_h, out_h, di_r, ones_v, di_t, acc,
                   isem, ssem):
        cid = lax.axis_index("c")
        sid = lax.axis_index("s")
        wid = sid * _NC + cid
        base = wid * e_per
        for i in range(_K // 16):
            ones_v[pl.ds(i * 16, 16)] = jnp.full((16,), 1.0, jnp.float32)

        def fire_idx(j, b):
            off = pl.multiple_of(e + base + j * _K, 8)
            pltpu.async_copy(ei_h.at[pl.ds(off, _K)], di_r.at[b],
                             isem.at[b])

        def wait_idx(j, b):
            off = pl.multiple_of(e + base + j * _K, 8)
            pltpu.make_async_copy(ei_h.at[pl.ds(off, _K)], di_r.at[b],
                                  isem.at[b]).wait()

        def fire_sc(b):
            pltpu.async_copy(ones_v, acc.at[di_r.at[b]], ssem.at[b],
                             add=True)

        def wait_sc(b):
            pltpu.make_async_copy(ones_v, acc.at[di_r.at[b]],
                                  ssem.at[b]).wait()

        fire_idx(0, 0)
        fire_idx(1, 1)
        _init_1d(zeros_h, acc, sid, 0)
        plsc.subcore_barrier()

        def step(j, b):
            @pl.when(j >= 3)
            def _():
                wait_sc((b + 3) % _DD)

            wait_idx(j, b)
            fire_sc(b)

            @pl.when(j + 2 < nfull)
            def _():
                fire_idx(j + 2, (b + 2) % _DD)

        def body(i, carry):
            for u in range(_DD):
                step(i * _DD + u, u)
            return carry

        lax.fori_loop(0, nfull // _DD, body, 0)
        for j in range(nfull - nfull % _DD, nfull):
            step(jnp.int32(j), j % _DD)
        for j in range(max(0, nfull - 3), nfull):
            wait_sc(j % _DD)
        if tail:
            off = pl.multiple_of(e + base + nfull * _K, 8)
            pltpu.sync_copy(ei_h.at[pl.ds(off, tail)], di_t)
            pltpu.sync_copy(ones_v.at[pl.ds(0, tail)], acc.at[di_t],
                            add=True)
        plsc.subcore_barrier()
        _init_1d(acc, out_h.at[cid], sid, 0)

    return deg_kernel(ei1, zeros_n)


def _sc_aggregate(table, ei1, zeros_nd, dis=None, zeros_n=None):
    n, d = table.shape
    e = ei1.shape[0] // 2
    e_per = e // _NW
    nfull = e_per // _K
    tail = e_per - nfull * _K
    with_s = dis is not None

    out_types = [jax.ShapeDtypeStruct((_NC, n, d), jnp.float32)]
    scratch = [
        pltpu.VMEM((_D, _K), jnp.int32),
        pltpu.VMEM((_D, _K), jnp.int32),
        pltpu.VMEM((2, _K, d), jnp.float32),
        pltpu.VMEM((tail,), jnp.int32) if tail else None,
        pltpu.VMEM((tail,), jnp.int32) if tail else None,
        pltpu.VMEM((tail, d), jnp.float32) if tail else None,
        pltpu.VMEM_SHARED((n, d), jnp.float32),
        pltpu.SemaphoreType.DMA((_D,)),
        pltpu.SemaphoreType.DMA((2,)),
        pltpu.SemaphoreType.DMA((2,)),
    ]
    if with_s:
        out_types.append(jax.ShapeDtypeStruct((_NC, n), jnp.float32))
        scratch += [
            pltpu.VMEM((2, _K), jnp.float32),
            pltpu.VMEM((tail,), jnp.float32) if tail else None,
            pltpu.VMEM_SHARED((n,), jnp.float32),
            pltpu.VMEM_SHARED((n,), jnp.float32),
            pltpu.SemaphoreType.DMA((2,)),
            pltpu.SemaphoreType.DMA((2,)),
        ]

    @functools.partial(
        pl.kernel,
        out_type=tuple(out_types),
        mesh=_sc_mesh(),
        scratch_types=scratch,
    )
    def agg_kernel(*refs):
        if with_s:
            (table_h, ei_h, zeros2_h, dis_h, zeros1_h,
             out_h, s_out_h,
             si_r, di_r, rows_r, si_t, di_t, rows_t, acc,
             isem, gsem, ssem,
             val_r, val_t, s_acc, dis_s, vgsem, sssem) = refs
        else:
            (table_h, ei_h, zeros2_h,
             out_h,
             si_r, di_r, rows_r, si_t, di_t, rows_t, acc,
             isem, gsem, ssem) = refs
        cid = lax.axis_index("c")
        sid = lax.axis_index("s")
        wid = sid * _NC + cid
        base = wid * e_per

        def fire_idx(j, b):
            off = pl.multiple_of(base + j * _K, 8)
            pltpu.async_copy(ei_h.at[pl.ds(off, _K)], si_r.at[b],
                             isem.at[b])
            pltpu.async_copy(ei_h.at[pl.ds(e + off, _K)], di_r.at[b],
                             isem.at[b])

        def wait_idx(j, b):
            off = pl.multiple_of(base + j * _K, 8)
            pltpu.make_async_copy(ei_h.at[pl.ds(off, _K)], si_r.at[b],
                                  isem.at[b]).wait()
            pltpu.make_async_copy(ei_h.at[pl.ds(e + off, _K)], di_r.at[b],
                                  isem.at[b]).wait()

        def fire_gather(ib, rb):
            pltpu.async_copy(table_h.at[si_r.at[ib]], rows_r.at[rb],
                             gsem.at[rb])
            if with_s:
                pltpu.async_copy(dis_s.at[di_r.at[ib]], val_r.at[rb],
                                 vgsem.at[rb])

        def wait_gather(ib, rb):
            pltpu.make_async_copy(table_h.at[si_r.at[ib]], rows_r.at[rb],
                                  gsem.at[rb]).wait()
            if with_s:
                pltpu.make_async_copy(dis_s.at[di_r.at[ib]],
                                      val_r.at[rb], vgsem.at[rb]).wait()

        def fire_sc(ib, rb):
            pltpu.async_copy(rows_r.at[rb], acc.at[di_r.at[ib]],
                             ssem.at[rb], add=True)
            if with_s:
                pltpu.async_copy(val_r.at[rb], s_acc.at[si_r.at[ib]],
                                 sssem.at[rb], add=True)

        def wait_sc(ib, rb):
            pltpu.make_async_copy(rows_r.at[rb], acc.at[di_r.at[ib]],
                                  ssem.at[rb]).wait()
            if with_s:
                pltpu.make_async_copy(val_r.at[rb], s_acc.at[si_r.at[ib]],
                                      sssem.at[rb]).wait()

        fire_idx(0, 0)
        fire_idx(1, 1)

        @pl.when(cid == 0)
        def _():
            _init_rows(table_h, acc, sid, n)

        @pl.when(cid == 1)
        def _():
            _init_rows(zeros2_h, acc, sid, n)

        if with_s:
            _init_1d(zeros1_h, s_acc, sid, _NS - 1)
            _init_1d(dis_h, dis_s, sid, _NS - 2)
        wait_idx(0, 0)
        fire_gather(0, 0)
        plsc.subcore_barrier()

        def step(j, ib, rb):
            @pl.when(j >= 1)
            def _():
                wait_sc((ib + _D - 1) % _D, 1 - rb)

            @pl.when(j + 1 < nfull)
            def _():
                wait_idx(j + 1, (ib + 1) % _D)
                fire_gather((ib + 1) % _D, 1 - rb)

            wait_gather(ib, rb)
            fire_sc(ib, rb)

            @pl.when(j + 2 < nfull)
            def _():
                fire_idx(j + 2, (ib + 2) % _D)

        def body(i, carry):
            for u in range(_D):
                step(i * _D + u, u, u % 2)
            return carry

        lax.fori_loop(0, nfull // _D, body, 0)
        for j in range(nfull - nfull % _D, nfull):
            step(jnp.int32(j), j % _D, j % 2)
        wait_sc((nfull - 1) % _D, (nfull - 1) % 2)
        if tail:
            off = pl.multiple_of(base + nfull * _K, 8)
            pltpu.sync_copy(ei_h.at[pl.ds(off, tail)], si_t)
            pltpu.sync_copy(ei_h.at[pl.ds(e + off, tail)], di_t)
            pltpu.async_copy(table_h.at[si_t], rows_t, gsem.at[0]).wait()
            pltpu.sync_copy(rows_t, acc.at[di_t], add=True)
            if with_s:
                pltpu.async_copy(dis_s.at[di_t], val_t,
                                 vgsem.at[0]).wait()
                pltpu.sync_copy(val_t, s_acc.at[si_t], add=True)
        plsc.subcore_barrier()

        _init_rows(acc, out_h.at[cid], sid, n)
        if with_s:
            _init_1d(s_acc, s_out_h.at[cid], sid, _NS - 1)

    if with_s:
        return agg_kernel(table, ei1, zeros_nd, dis, zeros_n)
    return agg_kernel(table, ei1, zeros_nd)[0]


def _tc_matmul(x, w1):
    n, d_in = x.shape
    d_h = w1.shape[1]
    nb = n // _ROWS_B

    def body(x_ref, w_ref, h_ref):
        h_ref[...] = jnp.dot(x_ref[...], w_ref[...],
                             preferred_element_type=jnp.float32)

    return pl.pallas_call(
        body,
        grid=(nb,),
        in_specs=[
            pl.BlockSpec((_ROWS_B, d_in), lambda i: (i, 0)),
            pl.BlockSpec((d_in, d_h), lambda i: (0, 0)),
        ],
        out_specs=pl.BlockSpec((_ROWS_B, d_h), lambda i: (i, 0)),
        out_shape=jax.ShapeDtypeStruct((n, d_h), jnp.float32),
    )(x, w1)


def _dis_col(cnt_ref):
    c = cnt_ref[...]
    return lax.rsqrt(c[:, 0:1] + c[:, 1:2] + 1.0)


def _tc_dis(cnt_p):
    n = cnt_p.shape[1]

    def body(cnt_ref, dis_ref):
        c = cnt_ref[...]
        dis_ref[...] = lax.rsqrt(c[0] + c[1] + 1.0)

    return pl.pallas_call(
        body,
        out_shape=jax.ShapeDtypeStruct((n,), jnp.float32),
    )(cnt_p)


def _tc_prep(cnt_t, h):
    n, d_h = h.shape
    nb = n // _ROWS_B

    def body(cnt_ref, h_ref, table_ref):
        table_ref[...] = _dis_col(cnt_ref) * h_ref[...]

    return pl.pallas_call(
        body,
        grid=(nb,),
        in_specs=[
            pl.BlockSpec((_ROWS_B, 2), lambda i: (i, 0)),
            pl.BlockSpec((_ROWS_B, d_h), lambda i: (i, 0)),
        ],
        out_specs=pl.BlockSpec((_ROWS_B, d_h), lambda i: (i, 0)),
        out_shape=jax.ShapeDtypeStruct((n, d_h), jnp.float32),
    )(cnt_t, h)


def _tc_layer(partials, cnt_t, b_row, w_next):
    n = partials.shape[1]
    d = partials.shape[2]
    d_next = w_next.shape[1]
    nb = n // _ROWS_B

    def body(p_ref, cnt_ref, b_ref, w_ref, out_ref):
        p = p_ref[0] + p_ref[1]
        dis = _dis_col(cnt_ref)
        h = jnp.maximum(dis * p + b_ref[...], 0.0)
        out_ref[...] = dis * jnp.dot(h, w_ref[...],
                                     preferred_element_type=jnp.float32)

    return pl.pallas_call(
        body,
        grid=(nb,),
        in_specs=[
            pl.BlockSpec((2, _ROWS_B, d), lambda i: (0, i, 0)),
            pl.BlockSpec((_ROWS_B, 2), lambda i: (i, 0)),
            pl.BlockSpec((1, d), lambda i: (0, 0)),
            pl.BlockSpec((d, d_next), lambda i: (0, 0)),
        ],
        out_specs=pl.BlockSpec((_ROWS_B, d_next), lambda i: (i, 0)),
        out_shape=jax.ShapeDtypeStruct((n, d_next), jnp.float32),
    )(partials, cnt_t, b_row, w_next)


def _tc_final(partials, cnt_t, s_t, b2_row, w3, b3_row, wc, bc_row):
    n = partials.shape[1]
    d = partials.shape[2]
    d_out = wc.shape[1]
    nb = n // _ROWS_B

    def body(p_ref, cnt_ref, s_ref, b2_ref, w3_ref, b3_ref, wc_ref,
             bc_ref, t_acc_ref, logits_ref):
        i = pl.program_id(0)
        dis = _dis_col(cnt_ref)
        p = p_ref[0] + p_ref[1]
        h2 = jnp.maximum(dis * p + b2_ref[...], 0.0)
        s = s_ref[:, 0:1] + s_ref[:, 1:2]
        c = dis * (dis + s)
        contrib = jnp.sum(c * h2, axis=0, keepdims=True)

        @pl.when(i == 0)
        def _():
            t_acc_ref[...] = jnp.zeros_like(t_acc_ref)

        t_acc_ref[...] += contrib

        @pl.when(i == nb - 1)
        def _():
            t = t_acc_ref[...] * (1.0 / n)
            g = jnp.dot(t, w3_ref[...],
                        preferred_element_type=jnp.float32) + b3_ref[...]
            logits_ref[...] = jnp.dot(g, wc_ref[...],
                                      preferred_element_type=jnp.float32) \
                + bc_ref[...]

    _, logits = pl.pallas_call(
        body,
        grid=(nb,),
        in_specs=[
            pl.BlockSpec((2, _ROWS_B, d), lambda i: (0, i, 0)),
            pl.BlockSpec((_ROWS_B, 2), lambda i: (i, 0)),
            pl.BlockSpec((_ROWS_B, 2), lambda i: (i, 0)),
            pl.BlockSpec((1, d), lambda i: (0, 0)),
            pl.BlockSpec((d, d), lambda i: (0, 0)),
            pl.BlockSpec((1, d), lambda i: (0, 0)),
            pl.BlockSpec((d, d_out), lambda i: (0, 0)),
            pl.BlockSpec((1, d_out), lambda i: (0, 0)),
        ],
        out_specs=[
            pl.BlockSpec((1, d), lambda i: (0, 0)),
            pl.BlockSpec((1, d_out), lambda i: (0, 0)),
        ],
        out_shape=[
            jax.ShapeDtypeStruct((1, d), jnp.float32),
            jax.ShapeDtypeStruct((1, d_out), jnp.float32),
        ],
    )(partials, cnt_t, s_t, b2_row, w3, b3_row, wc, bc_row)
    return logits


def kernel(x, edge_index, W1, b1, W2, b2, W3, b3, Wc, bc):
    n = x.shape[0]
    d_h = W1.shape[1]
    zeros_n = jnp.zeros((n,), jnp.float32)
    zeros_nd = jnp.zeros((n, d_h), jnp.float32)

    ei1 = edge_index.reshape(-1)
    h_raw = _tc_matmul(x, W1)
    cnt_p = _sc_degree(ei1, zeros_n)
    cnt_t = cnt_p.T
    dis1 = _tc_dis(cnt_p)
    table1 = _tc_prep(cnt_t, h_raw)
    p1 = _sc_aggregate(table1, ei1, zeros_nd)
    table2 = _tc_layer(p1, cnt_t, b1.reshape(1, -1), W2)
    p2, s_p = _sc_aggregate(table2, ei1, zeros_nd,
                            dis=dis1, zeros_n=zeros_n)
    logits = _tc_final(p2, cnt_t, s_p.T, b2.reshape(1, -1),
                       W3, b3.reshape(1, -1), Wc, bc.reshape(1, -1))
    return logits
